# baseline (speedup 1.0000x reference)
import jax
import jax.numpy as jnp
from jax import lax
from jax.experimental import pallas as pl
from jax.experimental.pallas import tpu as pltpu
from jax.experimental.pallas import tpu_sc as plsc

_SEQ_LEN = 16384
_VOCAB = 1000
_PAD = 0

_NC = 2
_NS = 16
_L = 16
_NW = _NC * _NS
_TPW = _SEQ_LEN // _NW
_CC = 128
_NCHUNK = _TPW // _CC
_VSPLIT = 496


def _one_hot_body(seq_hbm, vals_hbm, out_hbm, seq_v, vals_v, buf_a, buf_b,
                  sem_a, sem_b):
    wid = lax.axis_index("s") * _NC + lax.axis_index("c")
    base = wid * _TPW

    in_copies = [
        pltpu.async_copy(seq_hbm.at[pl.ds(base, _TPW)], seq_v, sem_a),
        pltpu.async_copy(vals_hbm.at[pl.ds(base, _TPW)], vals_v, sem_a),
    ]

    zeros16 = jnp.zeros((_L,), jnp.float32)
    lane = lax.iota(jnp.int32, _L)

    def zbody(r8, carry):
        for dr in range(8):
            for c0 in range(0, _CC, _L):
                off = pl.ds(c0, _L)
                buf_a[r8 * 8 + dr, off] = zeros16
                buf_b[r8 * 8 + dr, off] = zeros16
        return carry

    lax.fori_loop(0, _VSPLIT // 8, zbody, 0)
    for dr in range(_VSPLIT, _VOCAB - _VSPLIT):
        for c0 in range(0, _CC, _L):
            buf_b[dr, pl.ds(c0, _L)] = zeros16
    for cp in in_copies:
        cp.wait()

    def scatter(buf, lo, hi, c, value16):
        def body(g, inner):
            r0 = c * _CC + g * _L
            seq16 = seq_v[pl.ds(r0, _L)]
            v16 = value16 if value16 is not None else vals_v[pl.ds(r0, _L)]
            cols16 = g * _L + lane
            mask = (seq16 >= lo) & (seq16 < hi) if lo > 0 else (
                (seq16 != _PAD) & (seq16 < hi))
            plsc.store_scatter(buf, [seq16 - lo, cols16], v16, mask=mask)
            return inner

        lax.fori_loop(0, _CC // _L, body, 0)

    halves = (
        (buf_a, sem_a, 0, _VSPLIT,
         lambda c: out_hbm.at[pl.ds(0, _VSPLIT), pl.ds(base + c * _CC, _CC)]),
        (buf_b, sem_b, _VSPLIT, _VOCAB,
         lambda c: out_hbm.at[pl.ds(_VSPLIT, _VOCAB - _VSPLIT),
                              pl.ds(base + c * _CC, _CC)]),
    )

    for buf, sem, lo, hi, out_chunk in halves:
        scatter(buf, lo, hi, 0, None)
        pltpu.async_copy(buf, out_chunk(0), sem)

    def cbody(c, carry):
        for buf, sem, lo, hi, out_chunk in halves:
            pltpu.make_async_copy(buf, out_chunk(c - 1), sem).wait()
            scatter(buf, lo, hi, c - 1, zeros16)
            scatter(buf, lo, hi, c, None)
            pltpu.async_copy(buf, out_chunk(c), sem)
        return carry

    lax.fori_loop(1, _NCHUNK, cbody, 0)

    for buf, sem, lo, hi, out_chunk in halves:
        pltpu.make_async_copy(buf, out_chunk(_NCHUNK - 1), sem).wait()


@jax.jit
def kernel(sequence, vals):
    mesh = plsc.VectorSubcoreMesh(core_axis_name="c", subcore_axis_name="s")
    out_t = pl.kernel(
        _one_hot_body,
        mesh=mesh,
        compiler_params=pltpu.CompilerParams(
            needs_layout_passes=False, use_tc_tiling_on_sc=True),
        out_type=jax.ShapeDtypeStruct((_VOCAB, _SEQ_LEN), jnp.float32),
        scratch_types=[
            pltpu.VMEM((_TPW,), jnp.int32),
            pltpu.VMEM((_TPW,), jnp.float32),
            pltpu.VMEM((_VSPLIT, _CC), jnp.float32),
            pltpu.VMEM((_VOCAB - _VSPLIT, _CC), jnp.float32),
            pltpu.SemaphoreType.DMA,
            pltpu.SemaphoreType.DMA,
        ],
    )(sequence, vals)
    return out_t.T


# speedup vs baseline: 1.0128x; 1.0128x over previous
import jax
import jax.numpy as jnp
from jax import lax
from jax.experimental import pallas as pl
from jax.experimental.pallas import tpu as pltpu
from jax.experimental.pallas import tpu_sc as plsc

_SEQ_LEN = 16384
_VOCAB = 1000
_PAD = 0

_NC = 2
_NS = 16
_L = 16
_NW = _NC * _NS
_TPW = _SEQ_LEN // _NW
_CC = 128
_NCHUNK = _TPW // _CC


def _one_hot_body(seq_hbm, vals_hbm, out_hbm, seq_v, vals_v, buf, sem):
    wid = lax.axis_index("s") * _NC + lax.axis_index("c")
    base = wid * _TPW

    in_copies = [
        pltpu.async_copy(seq_hbm.at[pl.ds(base, _TPW)], seq_v, sem),
        pltpu.async_copy(vals_hbm.at[pl.ds(base, _TPW)], vals_v, sem),
    ]

    zeros16 = jnp.zeros((_L,), jnp.float32)
    lane = lax.iota(jnp.int32, _L)

    def zbody(r8, carry):
        for dr in range(8):
            for c0 in range(0, _CC, _L):
                buf[r8 * 8 + dr, pl.ds(c0, _L)] = zeros16
        return carry

    lax.fori_loop(0, _VOCAB // 8, zbody, 0)
    for cp in in_copies:
        cp.wait()

    def scatter(c, value16):
        def body(g, inner):
            r0 = c * _CC + g * _L
            seq16 = seq_v[pl.ds(r0, _L)]
            v16 = value16 if value16 is not None else vals_v[pl.ds(r0, _L)]
            cols16 = g * _L + lane
            plsc.store_scatter(buf, [seq16, cols16], v16,
                               mask=seq16 != _PAD)
            return inner

        lax.fori_loop(0, _CC // _L, body, 0)

    def cbody(c, carry):
        scatter(c, None)
        pltpu.sync_copy(buf, out_hbm.at[:, pl.ds(base + c * _CC, _CC)])
        scatter(c, zeros16)
        return carry

    lax.fori_loop(0, _NCHUNK, cbody, 0)


@jax.jit
def kernel(sequence, vals):
    mesh = plsc.VectorSubcoreMesh(core_axis_name="c", subcore_axis_name="s")
    out_t = pl.kernel(
        _one_hot_body,
        mesh=mesh,
        compiler_params=pltpu.CompilerParams(
            needs_layout_passes=False, use_tc_tiling_on_sc=True),
        out_type=jax.ShapeDtypeStruct((_VOCAB, _SEQ_LEN), jnp.float32),
        scratch_types=[
            pltpu.VMEM((_TPW,), jnp.int32),
            pltpu.VMEM((_TPW,), jnp.float32),
            pltpu.VMEM((_VOCAB, _CC), jnp.float32),
            pltpu.SemaphoreType.DMA,
        ],
    )(sequence, vals)
    return out_t.T
